# baseline (speedup 1.0000x reference)
import functools

import jax
import jax.numpy as jnp
from jax import lax
from jax.experimental import pallas as pl
from jax.experimental.pallas import tpu as pltpu
from jax.experimental.pallas import tpu_sc as plsc

N = 10000
H = 128
HH = H // 2
E = 320000
G = 512
NP = 10240
CH = 128
NBUF = 2
C = 158
EPAD = 16 * C * CH
STRIPE = NP // 16

BLK = 1000
NB = N // BLK
IB = 1024


_sc_mesh = plsc.VectorSubcoreMesh(core_axis_name="c", subcore_axis_name="s")


@functools.partial(
    pl.kernel,
    out_type=[
        jax.ShapeDtypeStruct((2, NP, HH), jnp.float32),
        jax.ShapeDtypeStruct((2, NP, 16), jnp.float32),
    ],
    mesh=_sc_mesh,
    scratch_types=[
        pltpu.VMEM((C, CH), jnp.int32),
        pltpu.VMEM((C, CH), jnp.int32),
        pltpu.VMEM((NBUF, CH, HH), jnp.float32),
        pltpu.VMEM((CH, 16), jnp.float32),
        pltpu.VMEM_SHARED((NP, HH), jnp.float32),
        pltpu.VMEM_SHARED((NP, 16), jnp.float32),
        [pltpu.SemaphoreType.DMA] * NBUF,
        [pltpu.SemaphoreType.DMA] * NBUF,
        pltpu.SemaphoreType.DMA,
    ],
    compiler_params=pltpu.CompilerParams(use_tc_tiling_on_sc=False),
)
def _sc_edges(x2_hbm, srcs_hbm, dsts_hbm, out_sum, out_deg,
              src_v, dst_v, rows_v, ones_v, acc_sh, deg_sh,
              sem_g, sem_s, sem_d):
    cid = lax.axis_index("c")
    sid = lax.axis_index("s")

    pltpu.sync_copy(srcs_hbm.at[cid, sid], src_v)
    pltpu.sync_copy(dsts_hbm.at[sid], dst_v)

    zeros16 = jnp.zeros((16,), jnp.float32)
    ones16 = jnp.full((16,), 1.0, jnp.float32)

    def _zrows(i, carry):
        r = i // (HH // 16)
        c = i % (HH // 16)
        rows_v[0, r, pl.ds(c * 16, 16)] = zeros16
        return carry

    lax.fori_loop(0, CH * HH // 16, _zrows, 0)

    def _fill(i, carry):
        ones_v[i, :] = zeros16
        return carry

    lax.fori_loop(0, CH, _fill, 0)
    for k in range(STRIPE // CH):
        pltpu.sync_copy(rows_v.at[0], acc_sh.at[pl.ds(sid * STRIPE + k * CH, CH)])
        pltpu.sync_copy(ones_v, deg_sh.at[pl.ds(sid * STRIPE + k * CH, CH)])

    def _fill2(i, carry):
        ones_v[i, :] = ones16
        return carry

    lax.fori_loop(0, CH, _fill2, 0)
    plsc.subcore_barrier()

    for b in range(NBUF):
        pltpu.async_copy(x2_hbm.at[src_v.at[b]], rows_v.at[b], sem_g[b])

    @pl.loop(0, C, step=NBUF)
    def _chunks(j):
        for b in range(NBUF):
            k = j + b
            pltpu.make_async_copy(x2_hbm.at[src_v.at[k]], rows_v.at[b],
                                  sem_g[b]).wait()
            pltpu.async_copy(rows_v.at[b], acc_sh.at[dst_v.at[k]],
                             sem_s[b], add=True)

            @pl.when(cid == (b % 2))
            def _():
                if b < 2:
                    @pl.when(j > 0)
                    def _():
                        pltpu.make_async_copy(ones_v, deg_sh.at[dst_v.at[k]],
                                              sem_d).wait()
                else:
                    pltpu.make_async_copy(ones_v, deg_sh.at[dst_v.at[k]],
                                          sem_d).wait()

                pltpu.async_copy(ones_v, deg_sh.at[dst_v.at[k]], sem_d, add=True)

        for b in range(NBUF):
            k = j + b
            pltpu.make_async_copy(rows_v.at[b], acc_sh.at[dst_v.at[k]],
                                  sem_s[b]).wait()

            @pl.when(k + NBUF < C)
            def _():
                pltpu.async_copy(x2_hbm.at[src_v.at[k + NBUF]], rows_v.at[b],
                                 sem_g[b])

    pltpu.make_async_copy(ones_v, deg_sh.at[dst_v.at[0]], sem_d).wait()
    plsc.subcore_barrier()

    pltpu.sync_copy(acc_sh.at[pl.ds(sid * STRIPE, STRIPE)],
                    out_sum.at[cid, pl.ds(sid * STRIPE, STRIPE)])
    pltpu.sync_copy(deg_sh.at[pl.ds(sid * STRIPE, STRIPE)],
                    out_deg.at[cid, pl.ds(sid * STRIPE, STRIPE)])


def _gru_pool_body(x_ref, xs_ref, degp_ref, b3_ref, wml_ref, bml_ref,
                   wih_ref, whh_ref, bih_ref, bhh_ref, gsum_ref, gcnt_ref):
    i = pl.program_id(0)
    x = x_ref[...]
    deg = jnp.maximum(degp_ref[0, :, 0] + degp_ref[1, :, 0], 1.0)
    xsum = jnp.concatenate([xs_ref[0], xs_ref[1]], axis=-1)
    xavg = xsum / deg[:, None]
    wml = wml_ref[...]
    bml = bml_ref[...]
    h0 = jnp.dot(x, wml, preferred_element_type=jnp.float32) + bml
    msg = jnp.dot(xavg, wml, preferred_element_type=jnp.float32) + bml
    gi = jnp.dot(msg, wih_ref[...], preferred_element_type=jnp.float32) + bih_ref[...]
    gh = jnp.dot(h0, whh_ref[...], preferred_element_type=jnp.float32) + bhh_ref[...]
    r = jax.nn.sigmoid(gi[:, :H] + gh[:, :H])
    z = jax.nn.sigmoid(gi[:, H:2 * H] + gh[:, H:2 * H])
    n = jnp.tanh(gi[:, 2 * H:] + r * gh[:, 2 * H:])
    h1 = (1.0 - z) * n + z * h0

    bid = b3_ref[0, 0, :]
    gids = lax.broadcasted_iota(jnp.int32, (G, BLK), 0)
    p = (gids == bid[None, :]).astype(jnp.float32)
    ps = jnp.dot(p, h1, preferred_element_type=jnp.float32)
    pc = jnp.sum(p, axis=1, keepdims=True)

    @pl.when(i == 0)
    def _():
        gsum_ref[...] = jnp.zeros_like(gsum_ref)
        gcnt_ref[...] = jnp.zeros_like(gcnt_ref)

    gsum_ref[...] += ps
    gcnt_ref[...] += jnp.broadcast_to(pc, (G, H))


_gru_pool = pl.pallas_call(
    _gru_pool_body,
    grid=(NB,),
    in_specs=[
        pl.BlockSpec((BLK, H), lambda i: (i, 0)),
        pl.BlockSpec((2, BLK, HH), lambda i: (0, i, 0)),
        pl.BlockSpec((2, BLK, 16), lambda i: (0, i, 0)),
        pl.BlockSpec((1, 1, BLK), lambda i: (i, 0, 0)),
        pl.BlockSpec((H, H), lambda i: (0, 0)),
        pl.BlockSpec((1, H), lambda i: (0, 0)),
        pl.BlockSpec((H, 3 * H), lambda i: (0, 0)),
        pl.BlockSpec((H, 3 * H), lambda i: (0, 0)),
        pl.BlockSpec((1, 3 * H), lambda i: (0, 0)),
        pl.BlockSpec((1, 3 * H), lambda i: (0, 0)),
    ],
    out_specs=[
        pl.BlockSpec((G, H), lambda i: (0, 0)),
        pl.BlockSpec((G, H), lambda i: (0, 0)),
    ],
    out_shape=[
        jax.ShapeDtypeStruct((G, H), jnp.float32),
        jax.ShapeDtypeStruct((G, H), jnp.float32),
    ],
)


def _fc_body(gsum_ref, gcnt_ref, wfct_ref, bfc_ref, out_ref):
    g = gsum_ref[...] / jnp.maximum(gcnt_ref[:, :1], 1.0)
    out_ref[...] = (jnp.dot(g, wfct_ref[...], preferred_element_type=jnp.float32)
                    + bfc_ref[...])


def _make_fc(ni):
    nblocks = -(-ni // IB)
    return pl.pallas_call(
        _fc_body,
        grid=(nblocks,),
        in_specs=[
            pl.BlockSpec((G, H), lambda j: (0, 0)),
            pl.BlockSpec((G, H), lambda j: (0, 0)),
            pl.BlockSpec((H, IB), lambda j: (0, j)),
            pl.BlockSpec((1, IB), lambda j: (0, j)),
        ],
        out_specs=pl.BlockSpec((G, IB), lambda j: (0, j)),
        out_shape=jax.ShapeDtypeStruct((G, ni), jnp.float32),
    )


def kernel(x, edge_index, batch, W_ml, b_ml, W_ih, W_hh, b_ih, b_hh, W_fc, b_fc):
    ni = W_fc.shape[0]
    src = edge_index[0].astype(jnp.int32)
    dst = edge_index[1].astype(jnp.int32)
    srcs = jnp.concatenate([src, jnp.zeros((EPAD - E,), jnp.int32)]).reshape(16, C, CH)
    srcs2 = jnp.stack([srcs, srcs + N])
    dsts = jnp.concatenate([dst, jnp.full((EPAD - E,), N, jnp.int32)]).reshape(16, C, CH)
    x2 = jnp.concatenate([x[:, :HH], x[:, HH:]], axis=0)
    batch3 = batch.astype(jnp.int32).reshape(NB, 1, BLK)

    xsum, deg = _sc_edges(x2, srcs2, dsts)

    gsum, gcnt = _gru_pool(
        x, xsum, deg, batch3,
        W_ml.T, b_ml.reshape(1, H),
        W_ih.T, W_hh.T, b_ih.reshape(1, 3 * H), b_hh.reshape(1, 3 * H),
    )
    scores = _make_fc(ni)(gsum, gcnt, W_fc.T, b_fc.reshape(1, ni))
    return scores


# speedup vs baseline: 1.2509x; 1.2509x over previous
import functools

import jax
import jax.numpy as jnp
from jax import lax
from jax.experimental import pallas as pl
from jax.experimental.pallas import tpu as pltpu
from jax.experimental.pallas import tpu_sc as plsc

N = 10000
H = 128
HH = H // 2
E = 320000
G = 512
NP = 10240
CH = 128
NBUF = 2
C = 158
EPAD = 16 * C * CH
STRIPE = NP // 16

BLK = 1000
NB = N // BLK
IB = 1024


_sc_mesh = plsc.VectorSubcoreMesh(core_axis_name="c", subcore_axis_name="s")


@functools.partial(
    pl.kernel,
    out_type=[
        jax.ShapeDtypeStruct((2, NP, HH), jnp.bfloat16),
        jax.ShapeDtypeStruct((2, NP, 16), jnp.float32),
    ],
    mesh=_sc_mesh,
    scratch_types=[
        pltpu.VMEM((C, CH), jnp.int32),
        pltpu.VMEM((C, CH), jnp.int32),
        pltpu.VMEM((NBUF, CH, HH), jnp.bfloat16),
        pltpu.VMEM((CH, 16), jnp.float32),
        pltpu.VMEM_SHARED((NP, HH), jnp.bfloat16),
        pltpu.VMEM_SHARED((NP, 16), jnp.float32),
        [pltpu.SemaphoreType.DMA] * NBUF,
        [pltpu.SemaphoreType.DMA] * NBUF,
        pltpu.SemaphoreType.DMA,
    ],
    compiler_params=pltpu.CompilerParams(use_tc_tiling_on_sc=False),
)
def _sc_edges(x2_hbm, srcs_hbm, dsts_hbm, out_sum, out_deg,
              src_v, dst_v, rows_v, ones_v, acc_sh, deg_sh,
              sem_g, sem_s, sem_d):
    cid = lax.axis_index("c")
    sid = lax.axis_index("s")

    pltpu.sync_copy(srcs_hbm.at[cid, sid], src_v)
    pltpu.sync_copy(dsts_hbm.at[sid], dst_v)

    zeros16 = jnp.zeros((16,), jnp.float32)
    ones16 = jnp.full((16,), 1.0, jnp.float32)
    zeros32b = jnp.zeros((32,), jnp.bfloat16)

    def _zrows(i, carry):
        r = i // (HH // 32)
        c = i % (HH // 32)
        rows_v[0, r, pl.ds(c * 32, 32)] = zeros32b
        return carry

    lax.fori_loop(0, CH * HH // 32, _zrows, 0)

    def _fill(i, carry):
        ones_v[i, :] = zeros16
        return carry

    lax.fori_loop(0, CH, _fill, 0)
    for k in range(STRIPE // CH):
        pltpu.sync_copy(rows_v.at[0], acc_sh.at[pl.ds(sid * STRIPE + k * CH, CH)])
        pltpu.sync_copy(ones_v, deg_sh.at[pl.ds(sid * STRIPE + k * CH, CH)])

    def _fill2(i, carry):
        ones_v[i, :] = ones16
        return carry

    lax.fori_loop(0, CH, _fill2, 0)
    plsc.subcore_barrier()

    for b in range(NBUF):
        pltpu.async_copy(x2_hbm.at[src_v.at[b]], rows_v.at[b], sem_g[b])

    @pl.loop(0, C, step=NBUF)
    def _chunks(j):
        for b in range(NBUF):
            k = j + b
            pltpu.make_async_copy(x2_hbm.at[src_v.at[k]], rows_v.at[b],
                                  sem_g[b]).wait()
            pltpu.async_copy(rows_v.at[b], acc_sh.at[dst_v.at[k]],
                             sem_s[b], add=True)

            @pl.when(cid == (b % 2))
            def _():
                if b < 2:
                    @pl.when(j > 0)
                    def _():
                        pltpu.make_async_copy(ones_v, deg_sh.at[dst_v.at[k]],
                                              sem_d).wait()
                else:
                    pltpu.make_async_copy(ones_v, deg_sh.at[dst_v.at[k]],
                                          sem_d).wait()

                pltpu.async_copy(ones_v, deg_sh.at[dst_v.at[k]], sem_d, add=True)

        for b in range(NBUF):
            k = j + b
            pltpu.make_async_copy(rows_v.at[b], acc_sh.at[dst_v.at[k]],
                                  sem_s[b]).wait()

            @pl.when(k + NBUF < C)
            def _():
                pltpu.async_copy(x2_hbm.at[src_v.at[k + NBUF]], rows_v.at[b],
                                 sem_g[b])

    pltpu.make_async_copy(ones_v, deg_sh.at[dst_v.at[0]], sem_d).wait()
    plsc.subcore_barrier()

    pltpu.sync_copy(acc_sh.at[pl.ds(sid * STRIPE, STRIPE)],
                    out_sum.at[cid, pl.ds(sid * STRIPE, STRIPE)])
    pltpu.sync_copy(deg_sh.at[pl.ds(sid * STRIPE, STRIPE)],
                    out_deg.at[cid, pl.ds(sid * STRIPE, STRIPE)])


def _gru_pool_body(x_ref, xs_ref, degp_ref, b3_ref, wml_ref, bml_ref,
                   wih_ref, whh_ref, bih_ref, bhh_ref, gsum_ref, gcnt_ref):
    i = pl.program_id(0)
    x = x_ref[...]
    deg = jnp.maximum(degp_ref[0, :, 0] + degp_ref[1, :, 0], 1.0)
    xsum = jnp.concatenate([xs_ref[0], xs_ref[1]], axis=-1).astype(jnp.float32)
    xavg = xsum / deg[:, None]
    wml = wml_ref[...]
    bml = bml_ref[...]
    h0 = jnp.dot(x, wml, preferred_element_type=jnp.float32) + bml
    msg = jnp.dot(xavg, wml, preferred_element_type=jnp.float32) + bml
    gi = jnp.dot(msg, wih_ref[...], preferred_element_type=jnp.float32) + bih_ref[...]
    gh = jnp.dot(h0, whh_ref[...], preferred_element_type=jnp.float32) + bhh_ref[...]
    r = jax.nn.sigmoid(gi[:, :H] + gh[:, :H])
    z = jax.nn.sigmoid(gi[:, H:2 * H] + gh[:, H:2 * H])
    n = jnp.tanh(gi[:, 2 * H:] + r * gh[:, 2 * H:])
    h1 = (1.0 - z) * n + z * h0

    bid = b3_ref[0, 0, :]
    gids = lax.broadcasted_iota(jnp.int32, (G, BLK), 0)
    p = (gids == bid[None, :]).astype(jnp.float32)
    ps = jnp.dot(p, h1, preferred_element_type=jnp.float32)
    pc = jnp.sum(p, axis=1, keepdims=True)

    @pl.when(i == 0)
    def _():
        gsum_ref[...] = jnp.zeros_like(gsum_ref)
        gcnt_ref[...] = jnp.zeros_like(gcnt_ref)

    gsum_ref[...] += ps
    gcnt_ref[...] += jnp.broadcast_to(pc, (G, H))


_gru_pool = pl.pallas_call(
    _gru_pool_body,
    grid=(NB,),
    in_specs=[
        pl.BlockSpec((BLK, H), lambda i: (i, 0)),
        pl.BlockSpec((2, BLK, HH), lambda i: (0, i, 0)),
        pl.BlockSpec((2, BLK, 16), lambda i: (0, i, 0)),
        pl.BlockSpec((1, 1, BLK), lambda i: (i, 0, 0)),
        pl.BlockSpec((H, H), lambda i: (0, 0)),
        pl.BlockSpec((1, H), lambda i: (0, 0)),
        pl.BlockSpec((H, 3 * H), lambda i: (0, 0)),
        pl.BlockSpec((H, 3 * H), lambda i: (0, 0)),
        pl.BlockSpec((1, 3 * H), lambda i: (0, 0)),
        pl.BlockSpec((1, 3 * H), lambda i: (0, 0)),
    ],
    out_specs=[
        pl.BlockSpec((G, H), lambda i: (0, 0)),
        pl.BlockSpec((G, H), lambda i: (0, 0)),
    ],
    out_shape=[
        jax.ShapeDtypeStruct((G, H), jnp.float32),
        jax.ShapeDtypeStruct((G, H), jnp.float32),
    ],
)


def _fc_body(gsum_ref, gcnt_ref, wfct_ref, bfc_ref, out_ref):
    g = gsum_ref[...] / jnp.maximum(gcnt_ref[:, :1], 1.0)
    out_ref[...] = (jnp.dot(g, wfct_ref[...], preferred_element_type=jnp.float32)
                    + bfc_ref[...])


def _make_fc(ni):
    nblocks = -(-ni // IB)
    return pl.pallas_call(
        _fc_body,
        grid=(nblocks,),
        in_specs=[
            pl.BlockSpec((G, H), lambda j: (0, 0)),
            pl.BlockSpec((G, H), lambda j: (0, 0)),
            pl.BlockSpec((H, IB), lambda j: (0, j)),
            pl.BlockSpec((1, IB), lambda j: (0, j)),
        ],
        out_specs=pl.BlockSpec((G, IB), lambda j: (0, j)),
        out_shape=jax.ShapeDtypeStruct((G, ni), jnp.float32),
    )


def kernel(x, edge_index, batch, W_ml, b_ml, W_ih, W_hh, b_ih, b_hh, W_fc, b_fc):
    ni = W_fc.shape[0]
    src = edge_index[0].astype(jnp.int32)
    dst = edge_index[1].astype(jnp.int32)
    srcs = jnp.concatenate([src, jnp.zeros((EPAD - E,), jnp.int32)]).reshape(16, C, CH)
    srcs2 = jnp.stack([srcs, srcs + N])
    dsts = jnp.concatenate([dst, jnp.full((EPAD - E,), N, jnp.int32)]).reshape(16, C, CH)
    x2 = jnp.concatenate([x[:, :HH], x[:, HH:]], axis=0).astype(jnp.bfloat16)
    batch3 = batch.astype(jnp.int32).reshape(NB, 1, BLK)

    xsum, deg = _sc_edges(x2, srcs2, dsts)

    gsum, gcnt = _gru_pool(
        x, xsum, deg, batch3,
        W_ml.T, b_ml.reshape(1, H),
        W_ih.T, W_hh.T, b_ih.reshape(1, 3 * H), b_hh.reshape(1, 3 * H),
    )
    scores = _make_fc(ni)(gsum, gcnt, W_fc.T, b_fc.reshape(1, ni))
    return scores


# speedup vs baseline: 1.2717x; 1.0166x over previous
import functools

import jax
import jax.numpy as jnp
from jax import lax
from jax.experimental import pallas as pl
from jax.experimental.pallas import tpu as pltpu
from jax.experimental.pallas import tpu_sc as plsc

N = 10000
H = 128
HH = H // 2
E = 320000
G = 512
NP = 10240
CH = 128
NBUF = 2
C = 158
EPAD = 16 * C * CH
STRIPE = NP // 16

BLK = 1000
NB = N // BLK
IB = 1024


_sc_mesh = plsc.VectorSubcoreMesh(core_axis_name="c", subcore_axis_name="s")


@functools.partial(
    pl.kernel,
    out_type=[
        jax.ShapeDtypeStruct((2, NP, HH), jnp.bfloat16),
        jax.ShapeDtypeStruct((2, NP, 16), jnp.float32),
    ],
    mesh=_sc_mesh,
    scratch_types=[
        pltpu.VMEM((C, CH), jnp.int32),
        pltpu.VMEM((C, CH), jnp.int32),
        pltpu.VMEM((NBUF, CH, HH), jnp.bfloat16),
        pltpu.VMEM((CH, 16), jnp.float32),
        pltpu.VMEM_SHARED((NP, HH), jnp.bfloat16),
        pltpu.VMEM_SHARED((NP, 16), jnp.float32),
        [pltpu.SemaphoreType.DMA] * NBUF,
        [pltpu.SemaphoreType.DMA] * NBUF,
        pltpu.SemaphoreType.DMA,
    ],
    compiler_params=pltpu.CompilerParams(use_tc_tiling_on_sc=False),
)
def _sc_edges(x2_hbm, srcs_hbm, dsts_hbm, out_sum, out_deg,
              src_v, dst_v, rows_v, ones_v, acc_sh, deg_sh,
              sem_g, sem_s, sem_d):
    cid = lax.axis_index("c")
    sid = lax.axis_index("s")

    pltpu.sync_copy(srcs_hbm.at[cid, sid], src_v)
    pltpu.sync_copy(dsts_hbm.at[sid], dst_v)

    zeros16 = jnp.zeros((16,), jnp.float32)
    ones16 = jnp.full((16,), 1.0, jnp.float32)
    zeros32b = jnp.zeros((32,), jnp.bfloat16)

    def _zrows(i, carry):
        r = i // (HH // 32)
        c = i % (HH // 32)
        rows_v[0, r, pl.ds(c * 32, 32)] = zeros32b
        return carry

    lax.fori_loop(0, CH * HH // 32, _zrows, 0)

    def _fill(i, carry):
        ones_v[i, :] = zeros16
        return carry

    lax.fori_loop(0, CH, _fill, 0)
    for k in range(STRIPE // CH):
        pltpu.sync_copy(rows_v.at[0], acc_sh.at[pl.ds(sid * STRIPE + k * CH, CH)])
        pltpu.sync_copy(ones_v, deg_sh.at[pl.ds(sid * STRIPE + k * CH, CH)])

    def _fill2(i, carry):
        ones_v[i, :] = ones16
        return carry

    lax.fori_loop(0, CH, _fill2, 0)
    plsc.subcore_barrier()

    for b in range(NBUF):
        pltpu.async_copy(x2_hbm.at[src_v.at[b]], rows_v.at[b], sem_g[b])

    @pl.loop(0, C, step=NBUF)
    def _chunks(j):
        for b in range(NBUF):
            k = j + b
            pltpu.make_async_copy(x2_hbm.at[src_v.at[k]], rows_v.at[b],
                                  sem_g[b]).wait()
            pltpu.async_copy(rows_v.at[b], acc_sh.at[dst_v.at[k]],
                             sem_s[b], add=True)

            @pl.when(cid == (b % 2))
            def _():
                if b < 2:
                    @pl.when(j > 0)
                    def _():
                        pltpu.make_async_copy(ones_v, deg_sh.at[dst_v.at[k]],
                                              sem_d).wait()
                else:
                    pltpu.make_async_copy(ones_v, deg_sh.at[dst_v.at[k]],
                                          sem_d).wait()

                pltpu.async_copy(ones_v, deg_sh.at[dst_v.at[k]], sem_d, add=True)

        for b in range(NBUF):
            k = j + b
            pltpu.make_async_copy(rows_v.at[b], acc_sh.at[dst_v.at[k]],
                                  sem_s[b]).wait()

            @pl.when(k + NBUF < C)
            def _():
                pltpu.async_copy(x2_hbm.at[src_v.at[k + NBUF]], rows_v.at[b],
                                 sem_g[b])

    pltpu.make_async_copy(ones_v, deg_sh.at[dst_v.at[0]], sem_d).wait()
    plsc.subcore_barrier()

    pltpu.sync_copy(acc_sh.at[pl.ds(sid * STRIPE, STRIPE)],
                    out_sum.at[cid, pl.ds(sid * STRIPE, STRIPE)])
    pltpu.sync_copy(deg_sh.at[pl.ds(sid * STRIPE, STRIPE)],
                    out_deg.at[cid, pl.ds(sid * STRIPE, STRIPE)])


def _gru_pool_body(x_ref, xs_ref, degp_ref, b3_ref, wml_ref, bml_ref,
                   wih_ref, whh_ref, bih_ref, bhh_ref, gsum_ref, gcnt_ref):
    i = pl.program_id(0)
    x = x_ref[...]
    deg = jnp.maximum(degp_ref[0, :, 0] + degp_ref[1, :, 0], 1.0)
    xsum = jnp.concatenate([xs_ref[0], xs_ref[1]], axis=-1).astype(jnp.float32)
    xavg = xsum / deg[:, None]
    def _dot_t(a, w):
        return lax.dot_general(a, w, (((1,), (1,)), ((), ())),
                               preferred_element_type=jnp.float32)

    wml = wml_ref[...]
    bml = bml_ref[...]
    h0 = _dot_t(x, wml) + bml
    msg = _dot_t(xavg, wml) + bml
    gi = _dot_t(msg, wih_ref[...]) + bih_ref[...]
    gh = _dot_t(h0, whh_ref[...]) + bhh_ref[...]
    r = jax.nn.sigmoid(gi[:, :H] + gh[:, :H])
    z = jax.nn.sigmoid(gi[:, H:2 * H] + gh[:, H:2 * H])
    n = jnp.tanh(gi[:, 2 * H:] + r * gh[:, 2 * H:])
    h1 = (1.0 - z) * n + z * h0

    bid = b3_ref[0, 0, :]
    gids = lax.broadcasted_iota(jnp.int32, (G, BLK), 0)
    p = (gids == bid[None, :]).astype(jnp.float32)
    ps = jnp.dot(p, h1, preferred_element_type=jnp.float32)
    pc = jnp.sum(p, axis=1, keepdims=True)

    @pl.when(i == 0)
    def _():
        gsum_ref[...] = jnp.zeros_like(gsum_ref)
        gcnt_ref[...] = jnp.zeros_like(gcnt_ref)

    gsum_ref[...] += ps
    gcnt_ref[...] += jnp.broadcast_to(pc, (G, H))


_gru_pool = pl.pallas_call(
    _gru_pool_body,
    grid=(NB,),
    in_specs=[
        pl.BlockSpec((BLK, H), lambda i: (i, 0)),
        pl.BlockSpec((2, BLK, HH), lambda i: (0, i, 0)),
        pl.BlockSpec((2, BLK, 16), lambda i: (0, i, 0)),
        pl.BlockSpec((1, 1, BLK), lambda i: (i, 0, 0)),
        pl.BlockSpec((H, H), lambda i: (0, 0)),
        pl.BlockSpec((1, H), lambda i: (0, 0)),
        pl.BlockSpec((3 * H, H), lambda i: (0, 0)),
        pl.BlockSpec((3 * H, H), lambda i: (0, 0)),
        pl.BlockSpec((1, 3 * H), lambda i: (0, 0)),
        pl.BlockSpec((1, 3 * H), lambda i: (0, 0)),
    ],
    out_specs=[
        pl.BlockSpec((G, H), lambda i: (0, 0)),
        pl.BlockSpec((G, H), lambda i: (0, 0)),
    ],
    out_shape=[
        jax.ShapeDtypeStruct((G, H), jnp.float32),
        jax.ShapeDtypeStruct((G, H), jnp.float32),
    ],
)


def _fc_body(gsum_ref, gcnt_ref, wfc_ref, bfc_ref, out_ref):
    g = gsum_ref[...] / jnp.maximum(gcnt_ref[:, :1], 1.0)
    out_ref[...] = (lax.dot_general(g, wfc_ref[...], (((1,), (1,)), ((), ())),
                                    preferred_element_type=jnp.float32)
                    + bfc_ref[...])


def _make_fc(ni):
    nblocks = -(-ni // IB)
    return pl.pallas_call(
        _fc_body,
        grid=(nblocks,),
        in_specs=[
            pl.BlockSpec((G, H), lambda j: (0, 0)),
            pl.BlockSpec((G, H), lambda j: (0, 0)),
            pl.BlockSpec((IB, H), lambda j: (j, 0)),
            pl.BlockSpec((1, IB), lambda j: (0, j)),
        ],
        out_specs=pl.BlockSpec((G, IB), lambda j: (0, j)),
        out_shape=jax.ShapeDtypeStruct((G, ni), jnp.float32),
    )


def kernel(x, edge_index, batch, W_ml, b_ml, W_ih, W_hh, b_ih, b_hh, W_fc, b_fc):
    ni = W_fc.shape[0]
    src = edge_index[0].astype(jnp.int32)
    dst = edge_index[1].astype(jnp.int32)
    srcs = jnp.concatenate([src, jnp.zeros((EPAD - E,), jnp.int32)]).reshape(16, C, CH)
    srcs2 = jnp.stack([srcs, srcs + N])
    dsts = jnp.concatenate([dst, jnp.full((EPAD - E,), N, jnp.int32)]).reshape(16, C, CH)
    x2 = jnp.concatenate([x[:, :HH], x[:, HH:]], axis=0).astype(jnp.bfloat16)
    batch3 = batch.astype(jnp.int32).reshape(NB, 1, BLK)

    xsum, deg = _sc_edges(x2, srcs2, dsts)

    gsum, gcnt = _gru_pool(
        x, xsum, deg, batch3,
        W_ml, b_ml.reshape(1, H),
        W_ih, W_hh, b_ih.reshape(1, 3 * H), b_hh.reshape(1, 3 * H),
    )
    scores = _make_fc(ni)(gsum, gcnt, W_fc, b_fc.reshape(1, ni))
    return scores
